# baseline (speedup 1.0000x reference)
import functools

import jax
import jax.numpy as jnp
from jax import lax
from jax.experimental import pallas as pl
from jax.experimental.pallas import tpu as pltpu
from jax.experimental.pallas import tpu_sc as plsc

NUM_EXPERTS = 8
TOP_K = 2
D_MODEL = 1024
D_FF = 2048
SEQ = 2048

T = SEQ
E = NUM_EXPERTS
P = T * TOP_K
BLK = 128
NT = P // BLK + E
R = NT * BLK

NC = 2
NS = 16
NW = NC * NS
RW = R // NW
CH = 32
NCH = RW // CH
TW = T // NW
CC = 32


def _routing_body(x_ref, wg_ref, pos_ref, wpair_ref, te_ref, ntot_ref):
    scores = jnp.dot(x_ref[...], wg_ref[...],
                     preferred_element_type=jnp.float32)
    lane = lax.broadcasted_iota(jnp.int32, (T, E), 1)
    m1 = jnp.max(scores, axis=-1, keepdims=True)
    a1 = jnp.min(jnp.where(scores == m1, lane, E), axis=-1, keepdims=True)
    oh1 = (lane == a1).astype(jnp.float32)
    s2 = jnp.where(lane == a1, -jnp.inf, scores)
    m2 = jnp.max(s2, axis=-1, keepdims=True)
    a2 = jnp.min(jnp.where(s2 == m2, lane, E), axis=-1, keepdims=True)
    oh2 = (lane == a2).astype(jnp.float32)
    e2 = jnp.exp(m2 - m1)
    w1 = 1.0 / (1.0 + e2)
    w2 = 1.0 - w1
    oh = jnp.concatenate([oh1, oh2], axis=0)
    wpair_ref[...] = jnp.concatenate([w1, w2], axis=0)

    sub = lax.broadcasted_iota(jnp.int32, (BLK, BLK), 0)
    ln2 = lax.broadcasted_iota(jnp.int32, (BLK, BLK), 1)
    ltri = (ln2 <= sub).astype(jnp.float32)
    carry = jnp.zeros((1, E), jnp.float32)
    ranks = []
    for b in range(P // BLK):
        blk = oh[b * BLK:(b + 1) * BLK, :]
        incl = jnp.dot(ltri, blk, preferred_element_type=jnp.float32)
        ranks.append(incl - blk + carry)
        carry = carry + jnp.sum(blk, axis=0, keepdims=True)
    rank = jnp.concatenate(ranks, axis=0)
    g = carry
    n = (g.astype(jnp.int32) + (BLK - 1)) // BLK
    esub = lax.broadcasted_iota(jnp.int32, (E, E), 0)
    elan = lax.broadcasted_iota(jnp.int32, (E, E), 1)
    mstrict = (esub < elan).astype(jnp.float32)
    ts = jnp.dot(n.astype(jnp.float32), mstrict,
                 preferred_element_type=jnp.float32)
    start = ts * float(BLK)
    rank_sel = jnp.sum(rank * oh, axis=1, keepdims=True)
    start_sel = jnp.sum(oh * start, axis=1, keepdims=True)
    pos_ref[...] = (start_sel + rank_sel).astype(jnp.int32)
    total = jnp.sum(n, axis=1, keepdims=True)
    ntot_ref[...] = total
    eye = (esub == elan).astype(jnp.float32)
    ts_col = jnp.sum(jnp.broadcast_to(ts, (E, E)) * eye, axis=1,
                     keepdims=True).astype(jnp.int32)
    jlane = lax.broadcasted_iota(jnp.int32, (1, NT), 1)
    je = jnp.minimum(jlane, total - 1)
    ind = (ts_col <= je).astype(jnp.int32)
    te_ref[...] = jnp.sum(ind, axis=0, keepdims=True) - 1


def _routing_call(xf, Wg):
    return pl.pallas_call(
        _routing_body,
        out_shape=(
            jax.ShapeDtypeStruct((P, 1), jnp.int32),
            jax.ShapeDtypeStruct((P, 1), jnp.float32),
            jax.ShapeDtypeStruct((1, NT), jnp.int32),
            jax.ShapeDtypeStruct((1, 1), jnp.int32),
        ),
    )(xf, Wg)


def _sc_mesh():
    return plsc.VectorSubcoreMesh(core_axis_name="c", subcore_axis_name="s")


_SC_PARAMS = pltpu.CompilerParams(needs_layout_passes=False)


def _dispatch_call(xf, pos, wpair):
    @functools.partial(
        pl.kernel,
        out_type=(
            jax.ShapeDtypeStruct((R, D_MODEL), jnp.float32),
            jax.ShapeDtypeStruct((R,), jnp.float32),
        ),
        mesh=_sc_mesh(),
        scratch_types=[
            pltpu.VMEM((P,), jnp.int32),
            pltpu.VMEM((P,), jnp.float32),
            pltpu.VMEM((NCH, CH), jnp.int32),
            pltpu.VMEM((RW,), jnp.float32),
            pltpu.VMEM((3, CH, D_MODEL), jnp.float32),
            pltpu.SemaphoreType.DMA,
            pltpu.SemaphoreType.DMA,
            pltpu.SemaphoreType.DMA,
        ],
        compiler_params=_SC_PARAMS,
    )
    def k(xf_hbm, pos_hbm, w_hbm, xs_hbm, wgt_hbm,
          pos_v, w_v, tok_loc, wgt_loc, rowbuf, sem0, sem1, sem2):
        wid = lax.axis_index("s") * NC + lax.axis_index("c")
        lo = wid * RW
        pltpu.sync_copy(pos_hbm, pos_v)
        pltpu.sync_copy(w_hbm, w_v)
        zi = jnp.zeros((16,), jnp.int32)
        zf = jnp.zeros((16,), jnp.float32)
        for r in range(NCH):
            for q in range(CH // 16):
                tok_loc[r, pl.ds(q * 16, 16)] = zi
        for i in range(RW // 16):
            wgt_loc[pl.ds(i * 16, 16)] = zf
        lane = lax.iota(jnp.int32, 16)

        def body(it, c):
            off = it * 16
            p16 = pos_v[pl.ds(off, 16)]
            w16 = w_v[pl.ds(off, 16)]
            tok = (lane + off) & (T - 1)
            rel = p16 - lo
            m = (rel >= 0) & (rel < RW)
            relc = jnp.where(m, rel, 0)
            rr = lax.div(relc, CH)
            rc = relc - rr * CH
            plsc.store_scatter(tok_loc, [rr, rc], tok, mask=m)
            plsc.store_scatter(wgt_loc, [relc], w16, mask=m)
            return c

        lax.fori_loop(0, P // 16, body, 0)
        pltpu.sync_copy(wgt_loc, wgt_hbm.at[pl.ds(lo, RW)])
        sems = [sem0, sem1, sem2]

        def start_g(ci):
            return pltpu.async_copy(
                xf_hbm.at[tok_loc.at[ci]], rowbuf.at[ci % 3], sems[ci % 3])

        gcp = [None] * NCH
        gcp[0] = start_g(0)
        gcp[1] = start_g(1)
        for ci in range(NCH):
            gcp[ci].wait()
            if ci + 2 < NCH:
                gcp[ci + 2] = start_g(ci + 2)
            pltpu.sync_copy(rowbuf.at[ci % 3],
                            xs_hbm.at[pl.ds(lo + ci * CH, CH)])

    return k(xf, pos, wpair)


def _ffn_body(te_ref, nt_ref, xs_ref, w1_ref, w2_ref, wg_ref, y_ref):
    j = pl.program_id(0)

    @pl.when(j < nt_ref[0])
    def _():
        xb = xs_ref[...].astype(jnp.bfloat16)
        h = jnp.dot(xb, w1_ref[0].astype(jnp.bfloat16),
                    preferred_element_type=jnp.float32)
        h = h * jax.nn.sigmoid(h)
        p = jnp.dot(h.astype(jnp.bfloat16), w2_ref[0].astype(jnp.bfloat16),
                    preferred_element_type=jnp.float32)
        y_ref[...] = p * wg_ref[...]


def _ffn_call(te, ntot, xs, W1, W2, wgt_col):
    grid_spec = pltpu.PrefetchScalarGridSpec(
        num_scalar_prefetch=2,
        grid=(NT,),
        in_specs=[
            pl.BlockSpec((BLK, D_MODEL), lambda j, te, nt: (j, 0)),
            pl.BlockSpec((1, D_MODEL, D_FF), lambda j, te, nt: (te[j], 0, 0)),
            pl.BlockSpec((1, D_FF, D_MODEL), lambda j, te, nt: (te[j], 0, 0)),
            pl.BlockSpec((BLK, 1), lambda j, te, nt: (j, 0)),
        ],
        out_specs=pl.BlockSpec((BLK, D_MODEL), lambda j, te, nt: (j, 0)),
    )
    return pl.pallas_call(
        _ffn_body,
        grid_spec=grid_spec,
        out_shape=jax.ShapeDtypeStruct((R, D_MODEL), jnp.float32),
    )(te, ntot, xs, W1, W2, wgt_col)


def _combine_call(Y, pos):
    @functools.partial(
        pl.kernel,
        out_type=jax.ShapeDtypeStruct((T, D_MODEL), jnp.float32),
        mesh=_sc_mesh(),
        scratch_types=[
            pltpu.VMEM((CC,), jnp.int32),
            pltpu.VMEM((CC,), jnp.int32),
            pltpu.VMEM((CC, D_MODEL), jnp.float32),
            pltpu.VMEM((CC, D_MODEL), jnp.float32),
            pltpu.SemaphoreType.DMA,
            pltpu.SemaphoreType.DMA,
        ],
        compiler_params=_SC_PARAMS,
    )
    def k(y_hbm, pos_hbm, out_hbm, idx_a, idx_b, bufa, bufb, sema, semb):
        wid = lax.axis_index("s") * NC + lax.axis_index("c")
        tbase = wid * TW
        for ci in range(TW // CC):
            t0 = tbase + ci * CC
            pltpu.sync_copy(pos_hbm.at[pl.ds(t0, CC)], idx_a)
            pltpu.sync_copy(pos_hbm.at[pl.ds(T + t0, CC)], idx_b)
            ca = pltpu.async_copy(y_hbm.at[idx_a], bufa, sema)
            cb = pltpu.async_copy(y_hbm.at[idx_b], bufb, semb)
            ca.wait()
            cb.wait()

            def row(r, c):
                for q in range(D_MODEL // 16):
                    sl = pl.ds(q * 16, 16)
                    bufa[r, sl] = bufa[r, sl] + bufb[r, sl]
                return c

            lax.fori_loop(0, CC, row, 0)
            pltpu.sync_copy(bufa, out_hbm.at[pl.ds(t0, CC)])

    return k(Y, pos)


@jax.jit
def kernel(x, Wg, W1, W2):
    orig_shape = x.shape
    xf = x.reshape(-1, x.shape[-1])

    pos, wpair, te, ntot = _routing_call(xf, Wg)
    pos = pos.reshape(P)
    wpair = wpair.reshape(P)
    te = te.reshape(NT)
    ntot = ntot.reshape(1)

    xs, wgt = _dispatch_call(xf, pos, wpair)
    Y = _ffn_call(te, ntot, xs, W1, W2, wgt.reshape(R, 1))
    out = _combine_call(Y, pos)
    return out.reshape(orig_shape)


# speedup vs baseline: 1.3053x; 1.3053x over previous
import functools

import jax
import jax.numpy as jnp
from jax import lax
from jax.experimental import pallas as pl
from jax.experimental.pallas import tpu as pltpu
from jax.experimental.pallas import tpu_sc as plsc

NUM_EXPERTS = 8
TOP_K = 2
D_MODEL = 1024
D_FF = 2048
SEQ = 2048

T = SEQ
E = NUM_EXPERTS
P = T * TOP_K
BLK = 128
NT = P // BLK + E
R = NT * BLK

NC = 2
NS = 16
NW = NC * NS
RW = R // NW
CH = 32
NCH = RW // CH
TW = T // NW
CC = 32


def _routing_body(x_ref, wg_ref, pos_ref, wpair_ref, te_ref, ntot_ref):
    scores = jnp.dot(x_ref[...], wg_ref[...],
                     preferred_element_type=jnp.float32)
    lane = lax.broadcasted_iota(jnp.int32, (T, E), 1)
    m1 = jnp.max(scores, axis=-1, keepdims=True)
    a1 = jnp.min(jnp.where(scores == m1, lane, E), axis=-1, keepdims=True)
    oh1 = (lane == a1).astype(jnp.float32)
    s2 = jnp.where(lane == a1, -jnp.inf, scores)
    m2 = jnp.max(s2, axis=-1, keepdims=True)
    a2 = jnp.min(jnp.where(s2 == m2, lane, E), axis=-1, keepdims=True)
    oh2 = (lane == a2).astype(jnp.float32)
    e2 = jnp.exp(m2 - m1)
    w1 = 1.0 / (1.0 + e2)
    w2 = 1.0 - w1
    oh = jnp.concatenate([oh1, oh2], axis=0)
    wpair_ref[...] = jnp.concatenate([w1, w2], axis=0)

    sub = lax.broadcasted_iota(jnp.int32, (BLK, BLK), 0)
    ln2 = lax.broadcasted_iota(jnp.int32, (BLK, BLK), 1)
    ltri = (ln2 <= sub).astype(jnp.float32)
    carry = jnp.zeros((1, E), jnp.float32)
    ranks = []
    for b in range(P // BLK):
        blk = oh[b * BLK:(b + 1) * BLK, :]
        incl = jnp.dot(ltri, blk, preferred_element_type=jnp.float32)
        ranks.append(incl - blk + carry)
        carry = carry + jnp.sum(blk, axis=0, keepdims=True)
    rank = jnp.concatenate(ranks, axis=0)
    g = carry
    n = (g.astype(jnp.int32) + (BLK - 1)) // BLK
    esub = lax.broadcasted_iota(jnp.int32, (E, E), 0)
    elan = lax.broadcasted_iota(jnp.int32, (E, E), 1)
    mstrict = (esub < elan).astype(jnp.float32)
    ts = jnp.dot(n.astype(jnp.float32), mstrict,
                 preferred_element_type=jnp.float32)
    start = ts * float(BLK)
    rank_sel = jnp.sum(rank * oh, axis=1, keepdims=True)
    start_sel = jnp.sum(oh * start, axis=1, keepdims=True)
    pos_ref[...] = (start_sel + rank_sel).astype(jnp.int32)
    total = jnp.sum(n, axis=1, keepdims=True)
    ntot_ref[...] = total
    eye = (esub == elan).astype(jnp.float32)
    ts_col = jnp.sum(jnp.broadcast_to(ts, (E, E)) * eye, axis=1,
                     keepdims=True).astype(jnp.int32)
    jlane = lax.broadcasted_iota(jnp.int32, (1, NT), 1)
    je = jnp.minimum(jlane, total - 1)
    ind = (ts_col <= je).astype(jnp.int32)
    te_ref[...] = jnp.sum(ind, axis=0, keepdims=True) - 1


def _routing_call(xf, Wg):
    return pl.pallas_call(
        _routing_body,
        out_shape=(
            jax.ShapeDtypeStruct((P, 1), jnp.int32),
            jax.ShapeDtypeStruct((P, 1), jnp.float32),
            jax.ShapeDtypeStruct((1, NT), jnp.int32),
            jax.ShapeDtypeStruct((1, 1), jnp.int32),
        ),
    )(xf, Wg)


def _sc_mesh():
    return plsc.VectorSubcoreMesh(core_axis_name="c", subcore_axis_name="s")


_SC_PARAMS = pltpu.CompilerParams(needs_layout_passes=False)


def _dispatch_call(pos, wpair):
    @functools.partial(
        pl.kernel,
        out_type=(
            jax.ShapeDtypeStruct((R,), jnp.int32),
            jax.ShapeDtypeStruct((R,), jnp.float32),
        ),
        mesh=_sc_mesh(),
        scratch_types=[
            pltpu.VMEM((P,), jnp.int32),
            pltpu.VMEM((P,), jnp.float32),
            pltpu.VMEM((RW,), jnp.int32),
            pltpu.VMEM((RW,), jnp.float32),
        ],
        compiler_params=_SC_PARAMS,
    )
    def k(pos_hbm, w_hbm, tok_hbm, wgt_hbm, pos_v, w_v, tok_loc, wgt_loc):
        wid = lax.axis_index("s") * NC + lax.axis_index("c")
        lo = wid * RW
        pltpu.sync_copy(pos_hbm, pos_v)
        pltpu.sync_copy(w_hbm, w_v)
        zi = jnp.zeros((16,), jnp.int32)
        zf = jnp.zeros((16,), jnp.float32)
        for i in range(RW // 16):
            tok_loc[pl.ds(i * 16, 16)] = zi
            wgt_loc[pl.ds(i * 16, 16)] = zf
        lane = lax.iota(jnp.int32, 16)

        def body(it, c):
            off = it * 16
            p16 = pos_v[pl.ds(off, 16)]
            w16 = w_v[pl.ds(off, 16)]
            tok = (lane + off) & (T - 1)
            rel = p16 - lo
            m = (rel >= 0) & (rel < RW)
            relc = jnp.where(m, rel, 0)
            plsc.store_scatter(tok_loc, [relc], tok, mask=m)
            plsc.store_scatter(wgt_loc, [relc], w16, mask=m)
            return c

        lax.fori_loop(0, P // 16, body, 0)
        pltpu.sync_copy(tok_loc, tok_hbm.at[pl.ds(lo, RW)])
        pltpu.sync_copy(wgt_loc, wgt_hbm.at[pl.ds(lo, RW)])

    return k(pos, wpair)


def _ffn_body(te_ref, nt_ref, xfb_ref, tok_ref, w1_ref, w2_ref, wg_ref, y_ref):
    j = pl.program_id(0)

    @pl.when(j < nt_ref[0])
    def _():
        lane_t = lax.broadcasted_iota(jnp.int32, (BLK, T), 1)
        sel = (lane_t == tok_ref[...]).astype(jnp.bfloat16)
        xv = jnp.dot(sel, xfb_ref[...], preferred_element_type=jnp.float32)
        xb = xv.astype(jnp.bfloat16)
        h = jnp.dot(xb, w1_ref[0].astype(jnp.bfloat16),
                    preferred_element_type=jnp.float32)
        h = h * jax.nn.sigmoid(h)
        p = jnp.dot(h.astype(jnp.bfloat16), w2_ref[0].astype(jnp.bfloat16),
                    preferred_element_type=jnp.float32)
        y_ref[...] = p * wg_ref[...]


def _ffn_call(te, ntot, xfb, tok_col, W1, W2, wgt_col):
    grid_spec = pltpu.PrefetchScalarGridSpec(
        num_scalar_prefetch=2,
        grid=(NT,),
        in_specs=[
            pl.BlockSpec((T, D_MODEL), lambda j, te, nt: (0, 0)),
            pl.BlockSpec((BLK, 1), lambda j, te, nt: (j, 0)),
            pl.BlockSpec((1, D_MODEL, D_FF), lambda j, te, nt: (te[j], 0, 0)),
            pl.BlockSpec((1, D_FF, D_MODEL), lambda j, te, nt: (te[j], 0, 0)),
            pl.BlockSpec((BLK, 1), lambda j, te, nt: (j, 0)),
        ],
        out_specs=pl.BlockSpec((BLK, D_MODEL), lambda j, te, nt: (j, 0)),
    )
    return pl.pallas_call(
        _ffn_body,
        grid_spec=grid_spec,
        out_shape=jax.ShapeDtypeStruct((R, D_MODEL), jnp.float32),
    )(te, ntot, xfb, tok_col, W1, W2, wgt_col)


def _combine_call(Y, pos):
    @functools.partial(
        pl.kernel,
        out_type=jax.ShapeDtypeStruct((T, D_MODEL), jnp.float32),
        mesh=_sc_mesh(),
        scratch_types=[
            pltpu.VMEM((CC,), jnp.int32),
            pltpu.VMEM((CC,), jnp.int32),
            pltpu.VMEM((CC, D_MODEL), jnp.float32),
            pltpu.VMEM((CC, D_MODEL), jnp.float32),
            pltpu.SemaphoreType.DMA,
            pltpu.SemaphoreType.DMA,
        ],
        compiler_params=_SC_PARAMS,
    )
    def k(y_hbm, pos_hbm, out_hbm, idx_a, idx_b, bufa, bufb, sema, semb):
        wid = lax.axis_index("s") * NC + lax.axis_index("c")
        tbase = wid * TW
        for ci in range(TW // CC):
            t0 = tbase + ci * CC
            pltpu.sync_copy(pos_hbm.at[pl.ds(t0, CC)], idx_a)
            pltpu.sync_copy(pos_hbm.at[pl.ds(T + t0, CC)], idx_b)
            ca = pltpu.async_copy(y_hbm.at[idx_a], bufa, sema)
            cb = pltpu.async_copy(y_hbm.at[idx_b], bufb, semb)
            ca.wait()
            cb.wait()

            def row(r, c):
                for q in range(D_MODEL // 16):
                    sl = pl.ds(q * 16, 16)
                    bufa[r, sl] = bufa[r, sl] + bufb[r, sl]
                return c

            lax.fori_loop(0, CC, row, 0)
            pltpu.sync_copy(bufa, out_hbm.at[pl.ds(t0, CC)])

    return k(Y, pos)


@jax.jit
def kernel(x, Wg, W1, W2):
    orig_shape = x.shape
    xf = x.reshape(-1, x.shape[-1])

    pos, wpair, te, ntot = _routing_call(xf, Wg)
    pos = pos.reshape(P)
    wpair = wpair.reshape(P)
    te = te.reshape(NT)
    ntot = ntot.reshape(1)

    tok, wgt = _dispatch_call(pos, wpair)
    Y = _ffn_call(te, ntot, xf.astype(jnp.bfloat16), tok.reshape(R, 1),
                  W1, W2, wgt.reshape(R, 1))
    out = _combine_call(Y, pos)
    return out.reshape(orig_shape)
